# baseline (speedup 1.0000x reference)
import functools

import jax
import jax.numpy as jnp
from jax import lax
from jax.experimental import pallas as pl
from jax.experimental.pallas import tpu as pltpu
from jax.experimental.pallas import tpu_sc as plsc

B = 50000
D = 128
E = 128
C = 16
S1 = 11

NC, NS = 2, 16
NW = NC * NS
RPS = 8
IPS = RPS * S1
IDXW = 96
NSTEPS = 196
BPW = NSTEPS * RPS
BPAD = NW * BPW


def _sc_gather_sum(features, idx_grp):
    mesh = plsc.VectorSubcoreMesh(core_axis_name="c", subcore_axis_name="s")

    @functools.partial(
        pl.kernel,
        mesh=mesh,
        out_type=jax.ShapeDtypeStruct((BPAD, D), jnp.float32),
        scratch_types=[
            pltpu.VMEM((NSTEPS, IDXW), jnp.int32),
            pltpu.VMEM((IDXW, D), jnp.float32),
            pltpu.VMEM((RPS, D), jnp.float32),
            pltpu.SemaphoreType.DMA,
        ],
    )
    def k(idx_hbm, feat_hbm, out_hbm, idx_slab, rows_v, acc_v, sem):
        wid = lax.axis_index("s") * NC + lax.axis_index("c")
        pltpu.sync_copy(idx_hbm.at[wid], idx_slab)

        def step(i, carry):
            pltpu.async_copy(feat_hbm.at[idx_slab.at[i]], rows_v, sem).wait()
            for r in range(RPS):
                for g in range(D // 16):
                    v = rows_v[r * S1, pl.ds(g * 16, 16)]
                    for j in range(1, S1):
                        v = v + rows_v[r * S1 + j, pl.ds(g * 16, 16)]
                    acc_v[r, pl.ds(g * 16, 16)] = v
            pltpu.sync_copy(acc_v, out_hbm.at[pl.ds(wid * BPW + i * RPS, RPS)])
            return carry

        lax.fori_loop(0, NSTEPS, step, 0)

    return k(idx_grp, features)


BLK = 1000


def _tc_head(sums, W0, W_cls):
    def body(x_ref, w0_ref, wc_ref, o_ref):
        x = x_ref[...] * (1.0 / S1)
        h = lax.dot_general(x, w0_ref[...], (((1,), (1,)), ((), ())),
                            preferred_element_type=jnp.float32)
        h = jnp.maximum(h, 0.0)
        s = lax.dot_general(h, wc_ref[...], (((1,), (1,)), ((), ())),
                            preferred_element_type=jnp.float32)
        o_ref[...] = jax.nn.sigmoid(s)

    return pl.pallas_call(
        body,
        grid=(B // BLK,),
        in_specs=[
            pl.BlockSpec((BLK, D), lambda i: (i, 0)),
            pl.BlockSpec((E, D), lambda i: (0, 0)),
            pl.BlockSpec((C, E), lambda i: (0, 0)),
        ],
        out_specs=pl.BlockSpec((BLK, C), lambda i: (i, 0)),
        out_shape=jax.ShapeDtypeStruct((B, C), jnp.float32),
    )(sums, W0, W_cls)


def kernel(features, W0, W_cls, nodes, neigh_idx):
    samp = jnp.concatenate([nodes[:, None], neigh_idx], axis=1)
    samp = jnp.pad(samp, ((0, BPAD - B), (0, 0)))
    idx_grp = samp.reshape(NW, NSTEPS, IPS)
    idx_grp = jnp.pad(idx_grp, ((0, 0), (0, 0), (0, IDXW - IPS)))
    sums = _sc_gather_sum(features, idx_grp)[:B]
    return _tc_head(sums, W0, W_cls)
